# baseline (speedup 1.0000x reference)
import jax
import jax.numpy as jnp
from jax import lax
from jax.experimental import pallas as pl
from jax.experimental.pallas import tpu as pltpu

_BLK = 2048
_CHUNK = 2048
_SUB = 1024


def _vfg_kernel(cf_ref, x_ref, out_ref, n_ref, n16_ref, m_ref):
    C, D = cf_ref.shape
    K = C // _BLK
    step = pl.program_id(0)

    @pl.when(step == 0)
    def _():
        cf = cf_ref[...]
        nrm2 = jnp.sum(cf * cf, axis=1, keepdims=True)
        n = cf * lax.rsqrt(jnp.maximum(nrm2, 1e-24))
        n_ref[...] = n
        n16_ref[...] = n.astype(jnp.bfloat16)
        m_ref[0, 0] = -jnp.inf

    @pl.when(step < K)
    def _():
        i = step
        base = i * _BLK
        nb = n16_ref[pl.ds(base, _BLK), :]

        mloc = jnp.float32(-jnp.inf)
        for a in range(_BLK // _SUB):
            width = _BLK - a * _SUB
            rb = n16_ref[pl.ds(base + a * _SUB, _SUB), :]
            cb = n16_ref[pl.ds(base + a * _SUB, width), :]
            s = lax.dot_general(rb, cb, (((1,), (1,)), ((), ())),
                                preferred_element_type=jnp.float32)
            rr = lax.broadcasted_iota(jnp.int32, (_SUB, width), 0)
            cc = lax.broadcasted_iota(jnp.int32, (_SUB, width), 1)
            s = jnp.where(rr == cc, -jnp.inf, s)
            mloc = jnp.maximum(mloc, jnp.max(s))

        def body(j, r):
            njb = n16_ref[pl.ds(j * _CHUNK, _CHUNK), :]
            s = lax.dot_general(nb, njb, (((1,), (1,)), ((), ())),
                                preferred_element_type=jnp.float32)
            return jnp.maximum(r, jnp.max(s, axis=0))

        r = lax.fori_loop(i + 1, C // _CHUNK, body,
                          jnp.full((_CHUNK,), -jnp.inf, jnp.float32))
        m_ref[0, 0] = jnp.maximum(m_ref[0, 0],
                                  jnp.maximum(mloc, jnp.max(r)))

    @pl.when(step == K)
    def _():
        n = n_ref[...]
        xv = x_ref[...]
        g = lax.dot_general(n, xv, (((0,), (0,)), ((), ())),
                            preferred_element_type=jnp.float32)
        d = jnp.sum(n * n, axis=1, keepdims=True)
        y = jnp.dot(n, g, preferred_element_type=jnp.float32) - d * xv
        m = m_ref[0, 0]
        scale = jnp.where(m > 0, 1.0 / jnp.where(m > 0, m, 1.0), 0.1)
        out_ref[...] = y * scale


def kernel(x, context_features, class_features):
    B, C, D = x.shape
    x2 = x.reshape(C, D)
    K = C // _BLK
    out = pl.pallas_call(
        _vfg_kernel,
        grid=(K + 1,),
        in_specs=[
            pl.BlockSpec((C, D), lambda i: (0, 0)),
            pl.BlockSpec((C, D), lambda i: (0, 0)),
        ],
        out_specs=pl.BlockSpec((C, D), lambda i: (0, 0)),
        out_shape=jax.ShapeDtypeStruct((C, D), jnp.float32),
        scratch_shapes=[
            pltpu.VMEM((C, D), jnp.float32),
            pltpu.VMEM((C, D), jnp.bfloat16),
            pltpu.SMEM((1, 1), jnp.float32),
        ],
        compiler_params=pltpu.CompilerParams(
            dimension_semantics=("arbitrary",),
        ),
    )(context_features, x2)
    return out.reshape(B, C, D)


# speedup vs baseline: 1.0952x; 1.0952x over previous
import jax
import jax.numpy as jnp
from jax import lax
from jax.experimental import pallas as pl
from jax.experimental.pallas import tpu as pltpu

_BLK = 2048
_CHUNK = 2048
_SUB = 256


def _vfg_kernel(cf_ref, x_ref, out_ref, n_ref, n16_ref, m_ref):
    C, D = cf_ref.shape
    K = C // _BLK
    step = pl.program_id(0)

    @pl.when(step == 0)
    def _():
        cf = cf_ref[...]
        nrm2 = jnp.sum(cf * cf, axis=1, keepdims=True)
        n = cf * lax.rsqrt(jnp.maximum(nrm2, 1e-24))
        n_ref[...] = n
        n16_ref[...] = n.astype(jnp.bfloat16)
        m_ref[0, 0] = -jnp.inf

    @pl.when(step < K)
    def _():
        i = step
        base = i * _BLK
        nb = n16_ref[pl.ds(base, _BLK), :]

        mloc = jnp.float32(-jnp.inf)
        for a in range(_BLK // _SUB):
            width = _BLK - a * _SUB
            rb = n16_ref[pl.ds(base + a * _SUB, _SUB), :]
            cb = n16_ref[pl.ds(base + a * _SUB, width), :]
            s = lax.dot_general(rb, cb, (((1,), (1,)), ((), ())),
                                preferred_element_type=jnp.float32)
            rr = lax.broadcasted_iota(jnp.int32, (_SUB, width), 0)
            cc = lax.broadcasted_iota(jnp.int32, (_SUB, width), 1)
            s = jnp.where(rr == cc, -jnp.inf, s)
            mloc = jnp.maximum(mloc, jnp.max(s))

        def body(j, r):
            njb = n16_ref[pl.ds(j * _CHUNK, _CHUNK), :]
            s = lax.dot_general(nb, njb, (((1,), (1,)), ((), ())),
                                preferred_element_type=jnp.float32)
            return jnp.maximum(r, jnp.max(s, axis=0))

        r = lax.fori_loop(i + 1, C // _CHUNK, body,
                          jnp.full((_CHUNK,), -jnp.inf, jnp.float32))
        m_ref[0, 0] = jnp.maximum(m_ref[0, 0],
                                  jnp.maximum(mloc, jnp.max(r)))

    @pl.when(step == K)
    def _():
        n = n_ref[...]
        xv = x_ref[...]
        g = lax.dot_general(n, xv, (((0,), (0,)), ((), ())),
                            preferred_element_type=jnp.float32)
        d = jnp.sum(n * n, axis=1, keepdims=True)
        y = jnp.dot(n, g, preferred_element_type=jnp.float32) - d * xv
        m = m_ref[0, 0]
        scale = jnp.where(m > 0, 1.0 / jnp.where(m > 0, m, 1.0), 0.1)
        out_ref[...] = y * scale


def kernel(x, context_features, class_features):
    B, C, D = x.shape
    x2 = x.reshape(C, D)
    K = C // _BLK
    out = pl.pallas_call(
        _vfg_kernel,
        grid=(K + 1,),
        in_specs=[
            pl.BlockSpec((C, D), lambda i: (0, 0)),
            pl.BlockSpec((C, D), lambda i: (0, 0)),
        ],
        out_specs=pl.BlockSpec((C, D), lambda i: (0, 0)),
        out_shape=jax.ShapeDtypeStruct((C, D), jnp.float32),
        scratch_shapes=[
            pltpu.VMEM((C, D), jnp.float32),
            pltpu.VMEM((C, D), jnp.bfloat16),
            pltpu.SMEM((1, 1), jnp.float32),
        ],
        compiler_params=pltpu.CompilerParams(
            dimension_semantics=("arbitrary",),
        ),
    )(context_features, x2)
    return out.reshape(B, C, D)
